# baseline (speedup 1.0000x reference)
import jax
import jax.numpy as jnp
from jax import lax
from jax.experimental import pallas as pl
from jax.experimental.pallas import tpu as pltpu
from jax.experimental.pallas import tpu_sc as plsc

NC = 2
NS = 16
L = 16
NW = NC * NS
B = 16384
D = 32
BPW = B // NW
ICH = 128
NCH = BPW // ICH


def _sc_body(i_hbm, j_hbm, k_hbm, u_hbm, v_hbm, out_hbm,
             idx_i, idx_j, idx_k, rows_u, rows_vj, rows_vk, out_v, sem):
    c = lax.axis_index("c")
    s = lax.axis_index("s")
    wid = s * NC + c
    base = wid * BPW
    row0 = wid * NCH

    pltpu.sync_copy(i_hbm.at[pl.ds(row0, NCH)], idx_i)
    pltpu.sync_copy(j_hbm.at[pl.ds(row0, NCH)], idx_j)
    pltpu.sync_copy(k_hbm.at[pl.ds(row0, NCH)], idx_k)

    copies = []
    for q in range(NCH):
        sl = pl.ds(q * ICH, ICH)
        copies.append(pltpu.async_copy(u_hbm.at[idx_i.at[q]], rows_u.at[sl], sem))
        copies.append(pltpu.async_copy(v_hbm.at[idx_j.at[q]], rows_vj.at[sl], sem))
        copies.append(pltpu.async_copy(v_hbm.at[idx_k.at[q]], rows_vk.at[sl], sem))
    for cp in copies:
        cp.wait()

    def chunk(cidx, carry):
        rbase = cidx * L
        rows = rbase + lax.iota(jnp.int32, L)
        accj = jnp.zeros((L,), jnp.float32)
        acck = jnp.zeros((L,), jnp.float32)
        for d in range(D):
            col = jnp.full((L,), d, jnp.int32)
            u = plsc.load_gather(rows_u, [rows, col])
            vj = plsc.load_gather(rows_vj, [rows, col])
            vk = plsc.load_gather(rows_vk, [rows, col])
            dj = u - vj
            dk = u - vk
            accj = accj + dj * dj
            acck = acck + dk * dk
        x = acck - accj
        out_v[pl.ds(rbase, L)] = 1.0 / (1.0 + jnp.exp(-x))
        return carry

    lax.fori_loop(0, BPW // L, chunk, 0)
    pltpu.sync_copy(out_v, out_hbm.at[pl.ds(base, BPW)])


@jax.jit
def kernel(i, j, k, U, V):
    i2 = i.reshape(B // ICH, ICH)
    j2 = j.reshape(B // ICH, ICH)
    k2 = k.reshape(B // ICH, ICH)
    mesh = plsc.VectorSubcoreMesh(
        core_axis_name="c", subcore_axis_name="s",
        num_cores=NC, num_subcores=NS)
    run = pl.kernel(
        _sc_body,
        out_type=jax.ShapeDtypeStruct((B,), jnp.float32),
        mesh=mesh,
        scratch_types=[
            pltpu.VMEM((NCH, ICH), jnp.int32),
            pltpu.VMEM((NCH, ICH), jnp.int32),
            pltpu.VMEM((NCH, ICH), jnp.int32),
            pltpu.VMEM((BPW, D), jnp.float32),
            pltpu.VMEM((BPW, D), jnp.float32),
            pltpu.VMEM((BPW, D), jnp.float32),
            pltpu.VMEM((BPW,), jnp.float32),
            pltpu.SemaphoreType.DMA,
        ],
        compiler_params=pltpu.CompilerParams(
            needs_layout_passes=False, use_tc_tiling_on_sc=False),
    )
    return run(i2, j2, k2, U, V)


# speedup vs baseline: 1.1559x; 1.1559x over previous
import jax
import jax.numpy as jnp
from jax import lax
from jax.experimental import pallas as pl
from jax.experimental.pallas import tpu as pltpu
from jax.experimental.pallas import tpu_sc as plsc

NC = 2
NS = 16
L = 16
NW = NC * NS
B = 16384
D = 32
BPW = B // NW
ICH = 128
NCH = BPW // ICH


def _sc_body(i_hbm, j_hbm, k_hbm, u_hbm, v_hbm, out_hbm,
             idx_i, idx_j, idx_k, rows_u, rows_vj, rows_vk, out_v, sem):
    c = lax.axis_index("c")
    s = lax.axis_index("s")
    wid = s * NC + c
    base = wid * BPW
    row0 = wid * NCH

    pltpu.sync_copy(i_hbm.at[pl.ds(row0, NCH)], idx_i)
    pltpu.sync_copy(j_hbm.at[pl.ds(row0, NCH)], idx_j)
    pltpu.sync_copy(k_hbm.at[pl.ds(row0, NCH)], idx_k)

    copies = []
    for q in range(NCH):
        sl = pl.ds(q * ICH, ICH)
        copies.append(pltpu.async_copy(u_hbm.at[idx_i.at[q]], rows_u.at[sl], sem))
        copies.append(pltpu.async_copy(v_hbm.at[idx_j.at[q]], rows_vj.at[sl], sem))
        copies.append(pltpu.async_copy(v_hbm.at[idx_k.at[q]], rows_vk.at[sl], sem))
    for cp in copies:
        cp.wait()

    lanes = lax.iota(jnp.int32, L)

    def chunk(cidx, carry):
        rbase = cidx * L
        rows = rbase + lanes
        accj = jnp.zeros((L,), jnp.float32)
        acck = jnp.zeros((L,), jnp.float32)
        for d in range(D):
            col = (lanes + d) & (D - 1)
            u = plsc.load_gather(rows_u, [rows, col])
            vj = plsc.load_gather(rows_vj, [rows, col])
            vk = plsc.load_gather(rows_vk, [rows, col])
            dj = u - vj
            dk = u - vk
            accj = accj + dj * dj
            acck = acck + dk * dk
        x = acck - accj
        out_v[pl.ds(rbase, L)] = 1.0 / (1.0 + jnp.exp(-x))
        return carry

    lax.fori_loop(0, BPW // L, chunk, 0)
    pltpu.sync_copy(out_v, out_hbm.at[pl.ds(base, BPW)])


@jax.jit
def kernel(i, j, k, U, V):
    i2 = i.reshape(B // ICH, ICH)
    j2 = j.reshape(B // ICH, ICH)
    k2 = k.reshape(B // ICH, ICH)
    mesh = plsc.VectorSubcoreMesh(
        core_axis_name="c", subcore_axis_name="s",
        num_cores=NC, num_subcores=NS)
    run = pl.kernel(
        _sc_body,
        out_type=jax.ShapeDtypeStruct((B,), jnp.float32),
        mesh=mesh,
        scratch_types=[
            pltpu.VMEM((NCH, ICH), jnp.int32),
            pltpu.VMEM((NCH, ICH), jnp.int32),
            pltpu.VMEM((NCH, ICH), jnp.int32),
            pltpu.VMEM((BPW, D), jnp.float32),
            pltpu.VMEM((BPW, D), jnp.float32),
            pltpu.VMEM((BPW, D), jnp.float32),
            pltpu.VMEM((BPW,), jnp.float32),
            pltpu.SemaphoreType.DMA,
        ],
        compiler_params=pltpu.CompilerParams(
            needs_layout_passes=False, use_tc_tiling_on_sc=False),
    )
    return run(i2, j2, k2, U, V)
